# baseline (device time: 48453 ns/iter reference)
import jax
import jax.numpy as jnp
from jax import lax
from jax.experimental import pallas as pl
from jax.experimental.pallas import tpu as pltpu

N_DEV = 16
N_EXPERTS = 32
CAPACITY = 102.0


def kernel(x, router_W, route_idx, expert_W):
    m, d = x.shape
    e_loc, _, h = expert_W.shape

    def body(x_ref, rw_ref, ri_ref, ew_ref, out_ref,
             myw, ew_all, ri_all, wsend, wrecv, rsend, rrecv):
        my = lax.axis_index("i")

        bar = pltpu.get_barrier_semaphore()
        for k in range(1, N_DEV):
            peer = lax.rem(my + k, N_DEV)
            pl.semaphore_signal(
                bar, inc=1,
                device_id=(peer,), device_id_type=pl.DeviceIdType.MESH,
            )
        pl.semaphore_wait(bar, N_DEV - 1)

        xb = x_ref[:].astype(jnp.bfloat16)
        route = ri_ref[:]
        myw[:] = ew_ref[:].astype(jnp.bfloat16)

        def mk(k):
            peer = lax.rem(my + k, N_DEV)
            kw = dict(device_id=(peer,), device_id_type=pl.DeviceIdType.MESH)
            return (
                pltpu.make_async_remote_copy(
                    src_ref=myw, dst_ref=ew_all.at[k],
                    send_sem=wsend.at[k], recv_sem=wrecv.at[k], **kw),
                pltpu.make_async_remote_copy(
                    src_ref=ri_ref, dst_ref=ri_all.at[k],
                    send_sem=rsend.at[k], recv_sem=rrecv.at[k], **kw),
            )

        for k in range(1, N_DEV):
            for r_ in mk(k):
                r_.start()

        ids = lax.broadcasted_iota(jnp.int32, (1, N_EXPERTS), 1)

        def chunk_contrib(w2, origin):
            e0 = e_loc * origin
            m0 = (route == e0).astype(jnp.bfloat16)
            m1 = (route == e0 + 1).astype(jnp.bfloat16)
            xm = jnp.concatenate([xb * m0, xb * m1], axis=1)
            w = w2.reshape(e_loc * d, h)
            return jnp.dot(xm, w, preferred_element_type=jnp.float32)

        def hist(rchunk):
            oh = (rchunk == ids).astype(jnp.float32)
            return jnp.sum(oh, axis=0, keepdims=True)

        acc = chunk_contrib(myw[:], my)
        oh_local = (route == ids).astype(jnp.float32)
        row = lax.broadcasted_iota(jnp.int32, (m, m), 0)
        col = lax.broadcasted_iota(jnp.int32, (m, m), 1)
        tril = (row > col).astype(jnp.float32)
        excl = jnp.dot(tril, oh_local, preferred_element_type=jnp.float32)
        prefix = jnp.zeros((1, N_EXPERTS), jnp.float32)

        for k in range(1, N_DEV):
            wr, rr = mk(k)
            wr.wait_recv()
            rr.wait_recv()
            origin = lax.rem(my - k + N_DEV, N_DEV)
            acc = acc + chunk_contrib(ew_all[k], origin)
            prefix = prefix + jnp.where(origin < my, hist(ri_all[k]), 0.0)

        for k in range(1, N_DEV):
            for r_ in mk(k):
                r_.wait_send()

        before = jnp.sum(
            oh_local * (excl + prefix), axis=1, keepdims=True
        )
        keep = (before < CAPACITY).astype(jnp.float32)
        out_ref[:] = acc * keep

    return pl.pallas_call(
        body,
        out_shape=jax.ShapeDtypeStruct((m, h), jnp.float32),
        in_specs=[pl.BlockSpec(memory_space=pltpu.VMEM)] * 4,
        out_specs=pl.BlockSpec(memory_space=pltpu.VMEM),
        scratch_shapes=[
            pltpu.VMEM((e_loc, d, h), jnp.bfloat16),
            pltpu.VMEM((N_DEV, e_loc, d, h), jnp.bfloat16),
            pltpu.VMEM((N_DEV, m, 1), jnp.int32),
            pltpu.SemaphoreType.DMA((N_DEV,)),
            pltpu.SemaphoreType.DMA((N_DEV,)),
            pltpu.SemaphoreType.DMA((N_DEV,)),
            pltpu.SemaphoreType.DMA((N_DEV,)),
        ],
        compiler_params=pltpu.CompilerParams(collective_id=0),
    )(x, router_W, route_idx, expert_W)


# device time: 34726 ns/iter; 1.3953x vs baseline; 1.3953x over previous
import jax
import jax.numpy as jnp
from jax import lax
from jax.experimental import pallas as pl
from jax.experimental.pallas import tpu as pltpu

N_DEV = 16
N_EXPERTS = 32
CAPACITY = 102.0
CW_HOPS = N_DEV // 2
CCW_HOPS = N_DEV - 1 - CW_HOPS
PACK_ROWS = 272


def kernel(x, router_W, route_idx, expert_W):
    m, d = x.shape
    e_loc, _, h = expert_W.shape

    def body(x_ref, rw_ref, ri_ref, ew_ref, out_ref,
             cw_buf, ccw_buf, cw_send, cw_recv, ccw_send, ccw_recv):
        my = lax.axis_index("i")
        left = lax.rem(my - 1 + N_DEV, N_DEV)
        right = lax.rem(my + 1, N_DEV)

        bar = pltpu.get_barrier_semaphore()
        for nbr in (left, right):
            pl.semaphore_signal(
                bar, inc=1,
                device_id=(nbr,), device_id_type=pl.DeviceIdType.MESH,
            )
        pl.semaphore_wait(bar, 2)

        xb = x_ref[:].astype(jnp.bfloat16)
        route = ri_ref[:]

        myw = ew_ref[:].astype(jnp.bfloat16).reshape(e_loc * d, h)
        my_route_row = route.astype(jnp.bfloat16).reshape(1, m)
        for buf in (cw_buf, ccw_buf):
            buf[0, 0:e_loc * d, :] = myw
            buf[0, e_loc * d:e_loc * d + 1, :] = my_route_row

        ids = lax.broadcasted_iota(jnp.int32, (1, N_EXPERTS), 1)
        ids_bf = ids.astype(jnp.bfloat16)

        def chunk_contrib(w, origin):
            e0 = e_loc * origin
            m0 = (route == e0).astype(jnp.bfloat16)
            m1 = (route == e0 + 1).astype(jnp.bfloat16)
            xm = jnp.concatenate([xb * m0, xb * m1], axis=1)
            return jnp.dot(xm, w, preferred_element_type=jnp.float32)

        def hist(route_row):
            oh = (route_row.reshape(m, 1) == ids_bf).astype(jnp.float32)
            return jnp.sum(oh, axis=0, keepdims=True)

        def mk(buf, snd, rcv, tgt, hop):
            return pltpu.make_async_remote_copy(
                src_ref=buf.at[hop], dst_ref=buf.at[hop + 1],
                send_sem=snd.at[hop], recv_sem=rcv.at[hop],
                device_id=(tgt,), device_id_type=pl.DeviceIdType.MESH,
            )

        def mk_cw(hop):
            return mk(cw_buf, cw_send, cw_recv, right, hop)

        def mk_ccw(hop):
            return mk(ccw_buf, ccw_send, ccw_recv, left, hop)

        mk_cw(0).start()
        mk_ccw(0).start()

        acc = chunk_contrib(myw, my)
        oh_local = (route == ids).astype(jnp.float32)
        row = lax.broadcasted_iota(jnp.int32, (m, m), 0)
        col = lax.broadcasted_iota(jnp.int32, (m, m), 1)
        tril = (row > col).astype(jnp.float32)
        excl = jnp.dot(tril, oh_local, preferred_element_type=jnp.float32)
        prefix = jnp.zeros((1, N_EXPERTS), jnp.float32)

        def absorb(buf, hop, origin):
            c = chunk_contrib(buf[hop + 1, 0:e_loc * d, :], origin)
            p = jnp.where(
                origin < my,
                hist(buf[hop + 1, e_loc * d:e_loc * d + 1, :]),
                0.0,
            )
            return c, p

        for hop in range(CW_HOPS):
            mk_cw(hop).wait_recv()
            if hop + 1 < CW_HOPS:
                mk_cw(hop + 1).start()
            if hop < CCW_HOPS:
                mk_ccw(hop).wait_recv()
                if hop + 1 < CCW_HOPS:
                    mk_ccw(hop + 1).start()

            c, p = absorb(cw_buf, hop, lax.rem(my - hop - 1 + N_DEV, N_DEV))
            acc, prefix = acc + c, prefix + p
            if hop < CCW_HOPS:
                c, p = absorb(ccw_buf, hop, lax.rem(my + hop + 1, N_DEV))
                acc, prefix = acc + c, prefix + p

        for hop in range(CW_HOPS):
            mk_cw(hop).wait_send()
        for hop in range(CCW_HOPS):
            mk_ccw(hop).wait_send()

        before = jnp.sum(
            oh_local * (excl + prefix), axis=1, keepdims=True
        )
        keep = (before < CAPACITY).astype(jnp.float32)
        out_ref[:] = acc * keep

    return pl.pallas_call(
        body,
        out_shape=jax.ShapeDtypeStruct((m, h), jnp.float32),
        in_specs=[pl.BlockSpec(memory_space=pltpu.VMEM)] * 4,
        out_specs=pl.BlockSpec(memory_space=pltpu.VMEM),
        scratch_shapes=[
            pltpu.VMEM((CW_HOPS + 1, PACK_ROWS, h), jnp.bfloat16),
            pltpu.VMEM((CCW_HOPS + 1, PACK_ROWS, h), jnp.bfloat16),
            pltpu.SemaphoreType.DMA((CW_HOPS,)),
            pltpu.SemaphoreType.DMA((CW_HOPS,)),
            pltpu.SemaphoreType.DMA((CCW_HOPS,)),
            pltpu.SemaphoreType.DMA((CCW_HOPS,)),
        ],
        compiler_params=pltpu.CompilerParams(collective_id=0),
    )(x, router_W, route_idx, expert_W)


# device time: 31145 ns/iter; 1.5557x vs baseline; 1.1150x over previous
import jax
import jax.numpy as jnp
from jax import lax
from jax.experimental import pallas as pl
from jax.experimental.pallas import tpu as pltpu

N_DEV = 16
N_EXPERTS = 32
CAPACITY = 102.0
GROUP = 4
STRIDE = N_DEV // GROUP
PACK_ROWS = 272


def kernel(x, router_W, route_idx, expert_W):
    m, d = x.shape
    e_loc, _, h = expert_W.shape
    wrows = e_loc * d

    def body(x_ref, rw_ref, ri_ref, ew_ref, out_ref,
             asm, cw1q, cw2q, ccw1q,
             p1_send, p1_recv, cw_send, cw_recv, ccw_send, ccw_recv):
        my = lax.axis_index("i")
        left = lax.rem(my - 1 + N_DEV, N_DEV)
        right = lax.rem(my + 1, N_DEV)

        bar = pltpu.get_barrier_semaphore()
        peer_offsets = [N_DEV - 1, 1, STRIDE, 2 * STRIDE, 3 * STRIDE]
        for off in peer_offsets:
            peer = lax.rem(my + off, N_DEV)
            pl.semaphore_signal(
                bar, inc=1,
                device_id=(peer,), device_id_type=pl.DeviceIdType.MESH,
            )
        pl.semaphore_wait(bar, len(peer_offsets))

        xb = x_ref[:].astype(jnp.bfloat16)
        route = ri_ref[:]

        myw = ew_ref[:].astype(jnp.bfloat16).reshape(wrows, h)
        asm[0, 0:wrows, :] = myw
        asm[0, wrows:wrows + 1, :] = route.astype(jnp.bfloat16).reshape(1, m)

        def mk_p1(mm):
            return pltpu.make_async_remote_copy(
                src_ref=asm.at[0], dst_ref=asm.at[(GROUP - mm) % GROUP],
                send_sem=p1_send.at[mm - 1],
                recv_sem=p1_recv.at[(GROUP - mm) - 1],
                device_id=(lax.rem(my + STRIDE * mm, N_DEV),),
                device_id_type=pl.DeviceIdType.MESH,
            )

        for mm in range(1, GROUP):
            mk_p1(mm).start()

        ids = lax.broadcasted_iota(jnp.int32, (1, N_EXPERTS), 1)
        ids_bf = ids.astype(jnp.bfloat16)

        def chunk_contrib(w, origin):
            e0 = e_loc * origin
            m0 = (route == e0).astype(jnp.bfloat16)
            m1 = (route == e0 + 1).astype(jnp.bfloat16)
            xm = jnp.concatenate([xb * m0, xb * m1], axis=1)
            return jnp.dot(xm, w, preferred_element_type=jnp.float32)

        def hist(route_row):
            oh = (route_row.reshape(m, 1) == ids_bf).astype(jnp.float32)
            return jnp.sum(oh, axis=0, keepdims=True)

        acc = chunk_contrib(myw, my)
        oh_local = (route == ids).astype(jnp.float32)
        row = lax.broadcasted_iota(jnp.int32, (m, m), 0)
        col = lax.broadcasted_iota(jnp.int32, (m, m), 1)
        tril = (row > col).astype(jnp.float32)
        excl = jnp.dot(tril, oh_local, preferred_element_type=jnp.float32)
        prefix = jnp.zeros((1, N_EXPERTS), jnp.float32)

        def absorb(buf, sub, origin):
            c = chunk_contrib(buf[sub, 0:wrows, :], origin)
            p = jnp.where(
                origin < my, hist(buf[sub, wrows:wrows + 1, :]), 0.0
            )
            return c, p

        for j in range(1, GROUP):
            mk_p1(GROUP - j).wait_recv()

        def mk_quad(src, dst, snd, rcv, idx, tgt):
            return pltpu.make_async_remote_copy(
                src_ref=src, dst_ref=dst,
                send_sem=snd.at[idx], recv_sem=rcv.at[idx],
                device_id=(tgt,), device_id_type=pl.DeviceIdType.MESH,
            )

        cw1 = mk_quad(asm, cw1q, cw_send, cw_recv, 0, right)
        ccw1 = mk_quad(asm, ccw1q, ccw_send, ccw_recv, 0, left)
        cw1.start()
        ccw1.start()

        for j in range(1, GROUP):
            c, p = absorb(asm, j, lax.rem(my + STRIDE * j, N_DEV))
            acc, prefix = acc + c, prefix + p

        cw1.wait_recv()
        cw2 = mk_quad(cw1q, cw2q, cw_send, cw_recv, 1, right)
        cw2.start()
        for j in range(GROUP):
            c, p = absorb(cw1q, j, lax.rem(my - 1 + STRIDE * j + N_DEV, N_DEV))
            acc, prefix = acc + c, prefix + p

        ccw1.wait_recv()
        for j in range(GROUP):
            c, p = absorb(ccw1q, j, lax.rem(my + 1 + STRIDE * j, N_DEV))
            acc, prefix = acc + c, prefix + p

        cw2.wait_recv()
        for j in range(GROUP):
            c, p = absorb(cw2q, j, lax.rem(my - 2 + STRIDE * j + N_DEV, N_DEV))
            acc, prefix = acc + c, prefix + p

        for mm in range(1, GROUP):
            mk_p1(mm).wait_send()
        cw1.wait_send()
        cw2.wait_send()
        ccw1.wait_send()

        before = jnp.sum(
            oh_local * (excl + prefix), axis=1, keepdims=True
        )
        keep = (before < CAPACITY).astype(jnp.float32)
        out_ref[:] = acc * keep

    quad = pltpu.VMEM((GROUP, PACK_ROWS, h), jnp.bfloat16)
    return pl.pallas_call(
        body,
        out_shape=jax.ShapeDtypeStruct((m, h), jnp.float32),
        in_specs=[pl.BlockSpec(memory_space=pltpu.VMEM)] * 4,
        out_specs=pl.BlockSpec(memory_space=pltpu.VMEM),
        scratch_shapes=[
            quad, quad, quad, quad,
            pltpu.SemaphoreType.DMA((GROUP - 1,)),
            pltpu.SemaphoreType.DMA((GROUP - 1,)),
            pltpu.SemaphoreType.DMA((2,)),
            pltpu.SemaphoreType.DMA((2,)),
            pltpu.SemaphoreType.DMA((1,)),
            pltpu.SemaphoreType.DMA((1,)),
        ],
        compiler_params=pltpu.CompilerParams(collective_id=0),
    )(x, router_W, route_idx, expert_W)
